# baseline (speedup 1.0000x reference)
import functools

import jax
import jax.numpy as jnp
from jax import lax
from jax.experimental import pallas as pl
from jax.experimental.pallas import tpu as pltpu, tpu_sc as plsc

_N, _D, _K = 65536, 64, 1024
_HALVES = 4
_NH = _N // _HALVES
_BLOCK = 1024
_NB = _NH // _BLOCK

_NC, _NS = 2, 16
_NW = _NC * _NS
_BPW = _NH // _NW
_CHUNK = 128
_NCHUNK = _BPW // _CHUNK


def _dist_body(zt_ref, c_ref, idx_ref, loss_ref, ids_ref):
    i = pl.program_id(0)

    @pl.when(i == 0)
    def _init_ids():
        ids_ref[...] = jax.lax.broadcasted_iota(
            jnp.int32, (_K, _BLOCK), 0).astype(jnp.float32)
    zt = zt_ref[...]
    c = c_ref[...]
    mT2 = jax.lax.dot_general(
        c * (-2.0), zt, dimension_numbers=(((1,), (0,)), ((), ())),
        preferred_element_type=jnp.float32)
    a = jnp.sum(zt * zt, axis=0)[None, :]
    c2 = jnp.sum(c * c, axis=1)[:, None]
    dT = (a + mT2) + c2
    t = dT
    r = _K
    while r > 8:
        h = r // 2
        t = jnp.minimum(t[:h], t[h:])
        r = h
    dmin = jnp.min(t, axis=0)
    cand = jnp.where(dT == dmin[None, :], ids_ref[...], jnp.float32(3.0e38))
    r = _K
    while r > 8:
        h = r // 2
        cand = jnp.minimum(cand[:h], cand[h:])
        r = h
    idx_ref[0, 0, :] = jnp.min(cand, axis=0).astype(jnp.int32)
    part = jnp.sum(dmin).reshape(1, 1)

    @pl.when(i == 0)
    def _init():
        loss_ref[...] = jnp.zeros((1, 1), jnp.float32)

    loss_ref[...] += part


def _tc_assign(zt, codebook):
    idx3, loss = pl.pallas_call(
        _dist_body,
        grid=(_NB,),
        in_specs=[
            pl.BlockSpec((_D, _BLOCK), lambda i: (0, i)),
            pl.BlockSpec((_K, _D), lambda i: (0, 0)),
        ],
        out_specs=[
            pl.BlockSpec((1, 1, _BLOCK), lambda i: (i, 0, 0)),
            pl.BlockSpec((1, 1), lambda i: (0, 0)),
        ],
        out_shape=[
            jax.ShapeDtypeStruct((_NB, 1, _BLOCK), jnp.int32),
            jax.ShapeDtypeStruct((1, 1), jnp.float32),
        ],
        scratch_shapes=[pltpu.VMEM((_K, _BLOCK), jnp.float32)],
    )(zt, codebook)
    return idx3.reshape(_NH), loss


_NBUF = 4
_AHEAD = 2


@functools.cache
def _make_sc_gather():
    @functools.partial(
        pl.kernel,
        mesh=plsc.VectorSubcoreMesh(core_axis_name="c", subcore_axis_name="s"),
        out_type=jax.ShapeDtypeStruct((_NH, 2 * _D), jnp.float32),
        scratch_types=(
            [pltpu.VMEM((_BPW,), jnp.int32)]
            + [pltpu.VMEM((_CHUNK, _D), jnp.float32) for _ in range(_NBUF)]
            + [pltpu.SemaphoreType.DMA for _ in range(2 * _NBUF)]
        ),
        compiler_params=pltpu.CompilerParams(use_tc_tiling_on_sc=False),
    )
    def _sc_gather(table_hbm, idx_hbm, out_hbm, idx_v, *bufs):
        rows = bufs[:_NBUF]
        gsem = bufs[_NBUF:2 * _NBUF]
        ssem = bufs[2 * _NBUF:]
        wid = lax.axis_index("s") * _NC + lax.axis_index("c")
        base = wid * _BPW
        out_base = base
        pltpu.sync_copy(idx_hbm.at[pl.ds(base, _BPW)], idx_v)
        gath = [None] * _NBUF
        scat = [None] * _NBUF
        for c in range(_NCHUNK + _AHEAD):
            if c < _NCHUNK:
                b = c % _NBUF
                if scat[b] is not None:
                    scat[b].wait()
                gath[b] = pltpu.async_copy(
                    table_hbm.at[idx_v.at[pl.ds(c * _CHUNK, _CHUNK)]],
                    rows[b], gsem[b])
            if c >= _AHEAD:
                cc = c - _AHEAD
                b = cc % _NBUF
                gath[b].wait()
                scat[b] = pltpu.async_copy(
                    rows[b],
                    out_hbm.at[pl.ds(out_base + cc * _CHUNK, _CHUNK), pl.ds(0, _D)],
                    ssem[b])
        for s in scat:
            if s is not None:
                s.wait()

    return _sc_gather


def kernel(z_e, codebook):
    sc_gather = _make_sc_gather()
    idx_halves, zq_halves, loss_total = [], [], None
    for h in range(_HALVES):
        zt_h = lax.slice_in_dim(z_e, h * _NH, (h + 1) * _NH, axis=0).T
        idx_h, loss_h = _tc_assign(zt_h, codebook)
        idx_halves.append(idx_h)
        zq_halves.append(sc_gather(codebook, idx_h))
        loss_total = loss_h if loss_total is None else loss_total + loss_h
    indices = jnp.concatenate(idx_halves)
    z_q = jnp.concatenate(zq_halves, axis=0)[:, :_D]
    commitment_loss = (loss_total[0, 0] / jnp.float32(_N * _D)).reshape(())
    return z_q, indices, commitment_loss


# speedup vs baseline: 1.0807x; 1.0807x over previous
import functools

import jax
import jax.numpy as jnp
from jax import lax
from jax.experimental import pallas as pl
from jax.experimental.pallas import tpu as pltpu, tpu_sc as plsc

_N, _D, _K = 65536, 64, 1024
_HALVES = 4
_NH = _N // _HALVES
_BLOCK = 1024
_NB = _NH // _BLOCK

_NC, _NS = 2, 16
_NW = _NC * _NS
_BPW = _NH // _NW
_CHUNK = 128
_NCHUNK = _BPW // _CHUNK


def _dist_body(zt_ref, c_ref, idx_ref, loss_ref, ids_ref):
    i = pl.program_id(0)

    @pl.when(i == 0)
    def _init_ids():
        ids_ref[...] = jax.lax.broadcasted_iota(
            jnp.int32, (_K, _BLOCK), 0).astype(jnp.float32)
    zt = zt_ref[...]
    c = c_ref[...]
    mT2 = jax.lax.dot_general(
        c * (-2.0), zt, dimension_numbers=(((1,), (0,)), ((), ())),
        preferred_element_type=jnp.float32)
    a = jnp.sum(zt * zt, axis=0)[None, :]
    c2 = jnp.sum(c * c, axis=1)[:, None]
    dT = (a + mT2) + c2
    t = dT
    r = _K
    while r > 8:
        h = r // 2
        t = jnp.minimum(t[:h], t[h:])
        r = h
    dmin = jnp.min(t, axis=0)
    cand = jnp.where(dT == dmin[None, :], ids_ref[...], jnp.float32(3.0e38))
    r = _K
    while r > 8:
        h = r // 2
        cand = jnp.minimum(cand[:h], cand[h:])
        r = h
    idx_ref[0, 0, :] = jnp.min(cand, axis=0).astype(jnp.int32)
    part = jnp.sum(dmin).reshape(1, 1)

    @pl.when(i == 0)
    def _init():
        loss_ref[...] = jnp.zeros((1, 1), jnp.float32)

    loss_ref[...] += part


def _tc_assign(zt, codebook):
    idx3, loss = pl.pallas_call(
        _dist_body,
        grid=(_NB,),
        in_specs=[
            pl.BlockSpec((_D, _BLOCK), lambda i: (0, i)),
            pl.BlockSpec((_K, _D), lambda i: (0, 0)),
        ],
        out_specs=[
            pl.BlockSpec((1, 1, _BLOCK), lambda i: (i, 0, 0)),
            pl.BlockSpec((1, 1), lambda i: (0, 0)),
        ],
        out_shape=[
            jax.ShapeDtypeStruct((_NB, 1, _BLOCK), jnp.int32),
            jax.ShapeDtypeStruct((1, 1), jnp.float32),
        ],
        scratch_shapes=[pltpu.VMEM((_K, _BLOCK), jnp.float32)],
    )(zt, codebook)
    return idx3.reshape(_NH), loss


_NBUF = 4
_AHEAD = 2


@functools.cache
def _make_sc_gather(half):
    @functools.partial(
        pl.kernel,
        mesh=plsc.VectorSubcoreMesh(core_axis_name="c", subcore_axis_name="s"),
        out_type=(),
        scratch_types=(
            [pltpu.VMEM((_BPW,), jnp.int32)]
            + [pltpu.VMEM((_CHUNK, _D), jnp.float32) for _ in range(_NBUF)]
            + [pltpu.SemaphoreType.DMA for _ in range(2 * _NBUF)]
        ),
        compiler_params=pltpu.CompilerParams(use_tc_tiling_on_sc=False),
    )
    def _sc_gather(table_hbm, idx_hbm, out_hbm, idx_v, *bufs):
        rows = bufs[:_NBUF]
        gsem = bufs[_NBUF:2 * _NBUF]
        ssem = bufs[2 * _NBUF:]
        wid = lax.axis_index("s") * _NC + lax.axis_index("c")
        base = wid * _BPW
        out_base = half * _NH + base
        pltpu.sync_copy(idx_hbm.at[pl.ds(base, _BPW)], idx_v)
        gath = [None] * _NBUF
        scat = [None] * _NBUF
        for c in range(_NCHUNK + _AHEAD):
            if c < _NCHUNK:
                b = c % _NBUF
                if scat[b] is not None:
                    scat[b].wait()
                gath[b] = pltpu.async_copy(
                    table_hbm.at[idx_v.at[pl.ds(c * _CHUNK, _CHUNK)]],
                    rows[b], gsem[b])
            if c >= _AHEAD:
                cc = c - _AHEAD
                b = cc % _NBUF
                gath[b].wait()
                scat[b] = pltpu.async_copy(
                    rows[b],
                    out_hbm.at[pl.ds(out_base + cc * _CHUNK, _CHUNK), pl.ds(0, _D)],
                    ssem[b])
        for s in scat:
            if s is not None:
                s.wait()

    return _sc_gather


def kernel(z_e, codebook):
    zq_ref = jax.new_ref(jnp.zeros((_N, 2 * _D), jnp.float32))
    idx_halves, loss_total = [], None
    for h in range(_HALVES):
        zt_h = lax.slice_in_dim(z_e, h * _NH, (h + 1) * _NH, axis=0).T
        idx_h, loss_h = _tc_assign(zt_h, codebook)
        idx_halves.append(idx_h)
        _make_sc_gather(h)(codebook, idx_h, zq_ref)
        loss_total = loss_h if loss_total is None else loss_total + loss_h
    indices = jnp.concatenate(idx_halves)
    z_q = jax.freeze(zq_ref)[:, :_D]
    commitment_loss = (loss_total[0, 0] / jnp.float32(_N * _D)).reshape(())
    return z_q, indices, commitment_loss


# speedup vs baseline: 1.1032x; 1.0208x over previous
import functools

import jax
import jax.numpy as jnp
from jax import lax
from jax.experimental import pallas as pl
from jax.experimental.pallas import tpu as pltpu, tpu_sc as plsc

_N, _D, _K = 65536, 64, 1024
_HALVES = 4
_NH = _N // _HALVES
_BLOCK = 1024
_NB = _NH // _BLOCK

_NC, _NS = 2, 16
_NW = _NC * _NS
_BPW = _NH // _NW
_CHUNK = 128
_NCHUNK = _BPW // _CHUNK


def _dist_body(zt_ref, c_ref, idx_ref, loss_ref, ids_ref):
    i = pl.program_id(0)

    @pl.when(i == 0)
    def _init_ids():
        ids_ref[...] = jax.lax.broadcasted_iota(
            jnp.int32, (_K, _BLOCK), 0).astype(jnp.float32)
    zt = zt_ref[...]
    c = c_ref[...]
    mT2 = jax.lax.dot_general(
        c * (-2.0), zt, dimension_numbers=(((1,), (0,)), ((), ())),
        preferred_element_type=jnp.float32)
    a = jnp.sum(zt * zt, axis=0)[None, :]
    c2 = jnp.sum(c * c, axis=1)[:, None]
    dT = (a + mT2) + c2
    t = dT
    r = _K
    while r > 8:
        h = r // 2
        t = jnp.minimum(t[:h], t[h:])
        r = h
    dmin = jnp.min(t, axis=0)
    cand = jnp.where(dT == dmin[None, :], ids_ref[...], jnp.float32(3.0e38))
    r = _K
    while r > 8:
        h = r // 2
        cand = jnp.minimum(cand[:h], cand[h:])
        r = h
    idx_ref[0, 0, :] = jnp.min(cand, axis=0).astype(jnp.int32)
    part = jnp.sum(dmin).reshape(1, 1)

    @pl.when(i == 0)
    def _init():
        loss_ref[...] = jnp.zeros((1, 1), jnp.float32)

    loss_ref[...] += part


def _tc_assign(zt, codebook):
    idx3, loss = pl.pallas_call(
        _dist_body,
        grid=(_NB,),
        in_specs=[
            pl.BlockSpec((_D, _BLOCK), lambda i: (0, i)),
            pl.BlockSpec((_K, _D), lambda i: (0, 0)),
        ],
        out_specs=[
            pl.BlockSpec((1, 1, _BLOCK), lambda i: (i, 0, 0)),
            pl.BlockSpec((1, 1), lambda i: (0, 0)),
        ],
        out_shape=[
            jax.ShapeDtypeStruct((_NB, 1, _BLOCK), jnp.int32),
            jax.ShapeDtypeStruct((1, 1), jnp.float32),
        ],
        scratch_shapes=[pltpu.VMEM((_K, _BLOCK), jnp.float32)],
    )(zt, codebook)
    return idx3.reshape(_NH), loss


_NBUF = 4
_AHEAD = 2


@functools.cache
def _make_sc_gather(half):
    @functools.partial(
        pl.kernel,
        mesh=plsc.VectorSubcoreMesh(core_axis_name="c", subcore_axis_name="s"),
        out_type=(),
        scratch_types=(
            [pltpu.VMEM((_BPW,), jnp.int32)]
            + [pltpu.VMEM((_CHUNK, _D), jnp.float32) for _ in range(_NBUF)]
            + [pltpu.SemaphoreType.DMA for _ in range(2 * _NBUF)]
        ),
        compiler_params=pltpu.CompilerParams(use_tc_tiling_on_sc=False),
    )
    def _sc_gather(table_hbm, idx_hbm, out_hbm, idx_v, *bufs):
        rows = bufs[:_NBUF]
        gsem = bufs[_NBUF:2 * _NBUF]
        ssem = bufs[2 * _NBUF:]
        wid = lax.axis_index("s") * _NC + lax.axis_index("c")
        base = wid * _BPW
        out_base = half * _NH + base
        pltpu.sync_copy(idx_hbm.at[pl.ds(base, _BPW)], idx_v)
        gath = [None] * _NBUF
        scat = [None] * _NBUF
        for c in range(_NCHUNK + _AHEAD):
            if c < _NCHUNK:
                b = c % _NBUF
                if scat[b] is not None:
                    scat[b].wait()
                gath[b] = pltpu.async_copy(
                    table_hbm.at[idx_v.at[pl.ds(c * _CHUNK, _CHUNK)]],
                    rows[b], gsem[b])
            if c >= _AHEAD:
                cc = c - _AHEAD
                b = cc % _NBUF
                gath[b].wait()
                scat[b] = pltpu.async_copy(
                    rows[b],
                    out_hbm.at[pl.ds(out_base + cc * _CHUNK, _CHUNK), pl.ds(0, _D)],
                    ssem[b])
        for s in scat:
            if s is not None:
                s.wait()

    return _sc_gather


@functools.cache
def _make_sc_fill():
    @functools.partial(
        pl.kernel,
        mesh=plsc.VectorSubcoreMesh(core_axis_name="c", subcore_axis_name="s"),
        out_type=jax.ShapeDtypeStruct((_N, 2 * _D), jnp.float32),
        scratch_types=[pltpu.VMEM((_CHUNK, 2 * _D), jnp.float32)],
    )
    def _sc_fill(out_hbm, buf):
        wid = lax.axis_index("s") * _NC + lax.axis_index("c")
        rows_per_w = _N // _NW
        for r in range(_CHUNK):
            for v in range(2 * _D // 16):
                buf[r, pl.ds(v * 16, 16)] = jnp.zeros((16,), jnp.float32)
        for c in range(rows_per_w // _CHUNK):
            pltpu.sync_copy(
                buf, out_hbm.at[pl.ds(wid * rows_per_w + c * _CHUNK, _CHUNK)])

    return _sc_fill


def kernel(z_e, codebook):
    zq_ref = jax.new_ref(_make_sc_fill()())
    idx_halves, loss_total = [], None
    for h in range(_HALVES):
        zt_h = lax.slice_in_dim(z_e, h * _NH, (h + 1) * _NH, axis=0).T
        idx_h, loss_h = _tc_assign(zt_h, codebook)
        idx_halves.append(idx_h)
        _make_sc_gather(h)(codebook, idx_h, zq_ref)
        loss_total = loss_h if loss_total is None else loss_total + loss_h
    indices = jnp.concatenate(idx_halves)
    z_q = jax.freeze(zq_ref)[:, :_D]
    commitment_loss = (loss_total[0, 0] / jnp.float32(_N * _D)).reshape(())
    return z_q, indices, commitment_loss


# speedup vs baseline: 1.1077x; 1.0041x over previous
import functools

import jax
import jax.numpy as jnp
from jax import lax
from jax.experimental import pallas as pl
from jax.experimental.pallas import tpu as pltpu, tpu_sc as plsc

_N, _D, _K = 65536, 64, 1024
_HALVES = 4
_NH = _N // _HALVES
_BLOCK = 1024
_NB = _NH // _BLOCK

_NC, _NS = 2, 16
_NW = _NC * _NS
_BPW = _NH // _NW
_CHUNK = 128
_NCHUNK = _BPW // _CHUNK


def _dist_body(zt_ref, c_ref, idx_ref, loss_ref, ids_ref):
    i = pl.program_id(0)

    @pl.when(i == 0)
    def _init_ids():
        ids_ref[...] = jax.lax.broadcasted_iota(
            jnp.int32, (_K, _BLOCK), 0).astype(jnp.float32)
    zt = zt_ref[...]
    c = c_ref[...]
    mT2 = jax.lax.dot_general(
        c * (-2.0), zt, dimension_numbers=(((1,), (0,)), ((), ())),
        preferred_element_type=jnp.float32)
    a = jnp.sum(zt * zt, axis=0)[None, :]
    c2 = jnp.sum(c * c, axis=1)[:, None]
    dT = (a + mT2) + c2
    t = dT
    r = _K
    while r > 8:
        h = r // 2
        t = jnp.minimum(t[:h], t[h:])
        r = h
    dmin = jnp.min(t, axis=0)
    cand = jnp.where(dT == dmin[None, :], ids_ref[...], jnp.float32(3.0e38))
    r = _K
    while r > 8:
        h = r // 2
        cand = jnp.minimum(cand[:h], cand[h:])
        r = h
    idx_ref[0, 0, :] = jnp.min(cand, axis=0).astype(jnp.int32)
    part = jnp.sum(dmin).reshape(1, 1)

    @pl.when(i == 0)
    def _init():
        loss_ref[...] = jnp.zeros((1, 1), jnp.float32)

    loss_ref[...] += part


def _tc_assign(zt, codebook):
    idx3, loss = pl.pallas_call(
        _dist_body,
        grid=(_NB,),
        in_specs=[
            pl.BlockSpec((_D, _BLOCK), lambda i: (0, i)),
            pl.BlockSpec((_K, _D), lambda i: (0, 0)),
        ],
        out_specs=[
            pl.BlockSpec((1, 1, _BLOCK), lambda i: (i, 0, 0)),
            pl.BlockSpec((1, 1), lambda i: (0, 0)),
        ],
        out_shape=[
            jax.ShapeDtypeStruct((_NB, 1, _BLOCK), jnp.int32),
            jax.ShapeDtypeStruct((1, 1), jnp.float32),
        ],
        scratch_shapes=[pltpu.VMEM((_K, _BLOCK), jnp.float32)],
    )(zt, codebook)
    return idx3.reshape(_NH), loss


_NBUF = 6
_AHEAD = 3


@functools.cache
def _make_sc_gather(half):
    @functools.partial(
        pl.kernel,
        mesh=plsc.VectorSubcoreMesh(core_axis_name="c", subcore_axis_name="s"),
        out_type=(),
        scratch_types=(
            [pltpu.VMEM((_BPW,), jnp.int32)]
            + [pltpu.VMEM((_CHUNK, _D), jnp.float32) for _ in range(_NBUF)]
            + [pltpu.SemaphoreType.DMA for _ in range(2 * _NBUF)]
        ),
        compiler_params=pltpu.CompilerParams(use_tc_tiling_on_sc=False),
    )
    def _sc_gather(table_hbm, idx_hbm, out_hbm, idx_v, *bufs):
        rows = bufs[:_NBUF]
        gsem = bufs[_NBUF:2 * _NBUF]
        ssem = bufs[2 * _NBUF:]
        wid = lax.axis_index("s") * _NC + lax.axis_index("c")
        base = wid * _BPW
        out_base = half * _NH + base
        pltpu.sync_copy(idx_hbm.at[pl.ds(base, _BPW)], idx_v)
        gath = [None] * _NBUF
        scat = [None] * _NBUF
        for c in range(_NCHUNK + _AHEAD):
            if c < _NCHUNK:
                b = c % _NBUF
                if scat[b] is not None:
                    scat[b].wait()
                gath[b] = pltpu.async_copy(
                    table_hbm.at[idx_v.at[pl.ds(c * _CHUNK, _CHUNK)]],
                    rows[b], gsem[b])
            if c >= _AHEAD:
                cc = c - _AHEAD
                b = cc % _NBUF
                gath[b].wait()
                scat[b] = pltpu.async_copy(
                    rows[b],
                    out_hbm.at[pl.ds(out_base + cc * _CHUNK, _CHUNK), pl.ds(0, _D)],
                    ssem[b])
        for s in scat:
            if s is not None:
                s.wait()

    return _sc_gather


@functools.cache
def _make_sc_fill():
    @functools.partial(
        pl.kernel,
        mesh=plsc.VectorSubcoreMesh(core_axis_name="c", subcore_axis_name="s"),
        out_type=jax.ShapeDtypeStruct((_N, 2 * _D), jnp.float32),
        scratch_types=[pltpu.VMEM((_CHUNK, 2 * _D), jnp.float32)],
    )
    def _sc_fill(out_hbm, buf):
        wid = lax.axis_index("s") * _NC + lax.axis_index("c")
        rows_per_w = _N // _NW
        for r in range(_CHUNK):
            for v in range(2 * _D // 16):
                buf[r, pl.ds(v * 16, 16)] = jnp.zeros((16,), jnp.float32)
        for c in range(rows_per_w // _CHUNK):
            pltpu.sync_copy(
                buf, out_hbm.at[pl.ds(wid * rows_per_w + c * _CHUNK, _CHUNK)])

    return _sc_fill


def kernel(z_e, codebook):
    zq_ref = jax.new_ref(_make_sc_fill()())
    idx_halves, loss_total = [], None
    for h in range(_HALVES):
        zt_h = lax.slice_in_dim(z_e, h * _NH, (h + 1) * _NH, axis=0).T
        idx_h, loss_h = _tc_assign(zt_h, codebook)
        idx_halves.append(idx_h)
        _make_sc_gather(h)(codebook, idx_h, zq_ref)
        loss_total = loss_h if loss_total is None else loss_total + loss_h
    indices = jnp.concatenate(idx_halves)
    z_q = jax.freeze(zq_ref)[:, :_D]
    commitment_loss = (loss_total[0, 0] / jnp.float32(_N * _D)).reshape(())
    return z_q, indices, commitment_loss
